# baseline (speedup 1.0000x reference)
import functools

import jax
import jax.numpy as jnp
from jax import lax
from jax.experimental import pallas as pl
from jax.experimental.pallas import tpu as pltpu
from jax.experimental.pallas import tpu_sc as plsc

_NC = 2
_NS = 16
_NW = _NC * _NS
_L = 16
_D = 64


@functools.partial(jax.jit, static_argnames=("bpw", "chunk"))
def _gather_call(idx, pairs, *, bpw, chunk):
    B = idx.shape[0]
    nchunk = bpw // chunk
    cb = chunk // _L
    mesh = plsc.VectorSubcoreMesh(core_axis_name="c", subcore_axis_name="s")

    @functools.partial(
        pl.kernel,
        out_type=jax.ShapeDtypeStruct((_D, B), jnp.float32),
        mesh=mesh,
        scratch_types=[
            pltpu.VMEM((bpw,), jnp.int32),
            pltpu.VMEM((bpw,), jnp.int32),
            pltpu.VMEM((2, chunk, 128), jnp.float32),
            pltpu.VMEM((2, _D, chunk), jnp.float32),
            pltpu.SemaphoreType.DMA,
            pltpu.SemaphoreType.DMA,
            pltpu.SemaphoreType.DMA,
            pltpu.SemaphoreType.DMA,
        ],
        compiler_params=pltpu.CompilerParams(needs_layout_passes=False),
    )
    def gather_kernel(idx_hbm, pairs_hbm, out_hbm, q_v, o_v, rows_v, t_v,
                      sg0, sg1, ss0, ss1):
        wid = lax.axis_index("s") * _NC + lax.axis_index("c")
        base = wid * bpw
        pltpu.sync_copy(idx_hbm.at[pl.ds(base, bpw)], q_v)

        lane = lax.broadcasted_iota(jnp.int32, (_L,), 0)

        def prep(i, carry):
            v = q_v[pl.ds(i * _L, _L)]
            o_v[pl.ds(i * _L, _L)] = lax.bitwise_and(v, 1) * _D
            q_v[pl.ds(i * _L, _L)] = lax.shift_right_logical(v, 1)
            return carry

        lax.fori_loop(0, bpw // _L, prep, 0, unroll=4)

        def xpose(k, rbuf, tbuf, obase):
            def body(it, carry):
                d = it // cb
                ib = it % cb
                rows = ib * _L + lane
                cols = o_v[pl.ds(obase + ib * _L, _L)] + d
                vals = plsc.load_gather(rbuf, [rows, cols])
                tbuf[d, pl.ds(ib * _L, _L)] = vals
                return carry

            lax.fori_loop(0, _D * cb, body, 0, unroll=8)

        sg = (sg0, sg1)
        ss = (ss0, ss1)
        gathers = [None, None]
        stores = [None, None]
        for i in range(nchunk + 1):
            if i < nchunk:
                b = i % 2
                gathers[b] = pltpu.async_copy(
                    pairs_hbm.at[q_v.at[pl.ds(i * chunk, chunk)]],
                    rows_v.at[b],
                    sg[b],
                )
            if i >= 1:
                j = i - 1
                bj = j % 2
                gathers[bj].wait()
                if stores[bj] is not None:
                    stores[bj].wait()
                xpose(j, rows_v.at[bj], t_v.at[bj], j * chunk)
                stores[bj] = pltpu.async_copy(
                    t_v.at[bj],
                    out_hbm.at[:, pl.ds(base + j * chunk, chunk)],
                    ss[bj],
                )
        for b in range(2):
            if stores[b] is not None:
                stores[b].wait()

    return gather_kernel(idx, pairs)


def kernel(x, mat):
    idx = x.reshape(-1)
    B = idx.shape[0]
    V = mat.shape[0]
    pairs = mat.reshape(V // 2, 128)
    bpw = B // _NW
    chunk = 256
    out_t = _gather_call(idx, pairs, bpw=bpw, chunk=chunk)
    return out_t.T


# speedup vs baseline: 1.7597x; 1.7597x over previous
import functools

import jax
import jax.numpy as jnp
from jax import lax
from jax.experimental import pallas as pl
from jax.experimental.pallas import tpu as pltpu
from jax.experimental.pallas import tpu_sc as plsc

_NC = 2
_NS = 16
_NW = _NC * _NS


@functools.partial(jax.jit, static_argnames=("bpw", "chunk"))
def _gather_call(idx, mat, *, bpw, chunk):
    B = idx.shape[0]
    D = mat.shape[1]
    nchunk = bpw // chunk
    mesh = plsc.VectorSubcoreMesh(core_axis_name="c", subcore_axis_name="s")

    @functools.partial(
        pl.kernel,
        out_type=jax.ShapeDtypeStruct((B, D), jnp.float32),
        mesh=mesh,
        scratch_types=[
            pltpu.VMEM((bpw,), jnp.int32),
            pltpu.VMEM((2, chunk, D), jnp.float32),
            pltpu.SemaphoreType.DMA,
            pltpu.SemaphoreType.DMA,
            pltpu.SemaphoreType.DMA,
            pltpu.SemaphoreType.DMA,
        ],
        compiler_params=pltpu.CompilerParams(use_tc_tiling_on_sc=False),
    )
    def gather_kernel(idx_hbm, mat_hbm, out_hbm, idx_v, rows_v, sg0, sg1, ss0, ss1):
        wid = lax.axis_index("s") * _NC + lax.axis_index("c")
        base = wid * bpw
        pltpu.sync_copy(idx_hbm.at[pl.ds(base, bpw)], idx_v)

        sg = (sg0, sg1)
        ss = (ss0, ss1)
        gathers = [None, None]
        stores = [None, None]
        for i in range(nchunk + 1):
            if i < nchunk:
                b = i % 2
                if stores[b] is not None:
                    stores[b].wait()
                    stores[b] = None
                gathers[b] = pltpu.async_copy(
                    mat_hbm.at[idx_v.at[pl.ds(i * chunk, chunk)]],
                    rows_v.at[b],
                    sg[b],
                )
            if i >= 1:
                j = i - 1
                bj = j % 2
                gathers[bj].wait()
                stores[bj] = pltpu.async_copy(
                    rows_v.at[bj],
                    out_hbm.at[pl.ds(base + j * chunk, chunk)],
                    ss[bj],
                )
        for b in range(2):
            if stores[b] is not None:
                stores[b].wait()

    return gather_kernel(idx, mat)


def _pick_chunk(bpw, d):
    budget = 430 * 1024 - bpw * 4
    best = 8
    c = 8
    while c <= bpw:
        if bpw % c == 0 and 2 * c * d * 4 <= budget:
            best = c
        c += 8
    return best


def kernel(x, mat):
    idx = x.reshape(-1)
    B = idx.shape[0]
    D = mat.shape[1]
    bpw = B // _NW
    chunk = _pick_chunk(bpw, D)
    return _gather_call(idx, mat, bpw=bpw, chunk=chunk)
